# Initial kernel scaffold; baseline (speedup 1.0000x reference)
import jax
import jax.numpy as jnp
from jax.experimental import pallas as pl


def kernel(z, xyz, nbrs, charge, spin, num_atoms, emb_z, q_vec, s_vec, W_g, W1, W2, W_out, w_read):
    raise NotImplementedError("write your pallas kernel here")


# speedup vs baseline: 1.2308x; 1.2308x over previous
import jax
import jax.numpy as jnp
from jax.experimental import pallas as pl
from jax.scipy.special import gammaln

N = 10000
B = 100
E = 160000
F = 128
MAXZ = 87
K = 20
NCONV = 3
RCUT = 5.0
GAMMA = 0.5


def _silu(h):
    return h * jax.nn.sigmoid(h)


def _silu_prime(h):
    s = jax.nn.sigmoid(h)
    return s * (1.0 + h * (1.0 - s))


def kernel(z, xyz, nbrs, charge, spin, num_atoms, emb_z, q_vec, s_vec, W_g, W1, W2, W_out, w_read):
    mol_id = jnp.repeat(jnp.arange(B), N // B)
    src = nbrs[:, 0]
    dst = nbrs[:, 1]

    x0 = jnp.take(emb_z, z, axis=0) \
        + charge[mol_id][:, None] * q_vec[None, :] \
        + spin[mol_id][:, None] * s_vec[None, :]

    r = jnp.take(xyz, dst, axis=0) - jnp.take(xyz, src, axis=0)
    d = jnp.sqrt(jnp.sum(r * r, axis=-1) + 1e-12)
    valid = (src != dst).astype(jnp.float32)
    inside = (d < RCUT).astype(jnp.float32)
    fc = 0.5 * (jnp.cos(jnp.pi * d / RCUT) + 1.0) * inside * valid
    fcp = -0.5 * (jnp.pi / RCUT) * jnp.sin(jnp.pi * d / RCUT) * inside * valid

    u = jnp.exp(-GAMMA * d)
    kk = jnp.arange(K, dtype=jnp.float32)
    logbin = gammaln(float(K)) - gammaln(kk + 1.0) - gammaln(float(K) - kk)
    cu = jnp.clip(u, 1e-10, 1.0)
    c1u = jnp.clip(1.0 - u, 1e-10, 1.0)
    lu = jnp.log(cu)
    l1u = jnp.log(c1u)
    bern = jnp.exp(logbin[None, :] + kk[None, :] * lu[:, None]
                   + (K - 1.0 - kk)[None, :] * l1u[:, None])
    basis = bern * fc[:, None]

    xs = [x0]
    hs = []
    gs = []
    f = jnp.zeros_like(x0)
    x = x0
    for t in range(NCONV):
        g = basis @ W_g[t]
        gs.append(g)
        m = jax.ops.segment_sum(jnp.take(x, src, axis=0) * g, dst, num_segments=N)
        h = (x + m) @ W1[t]
        hs.append(h)
        x = x + _silu(h) @ W2[t]
        xs.append(x)
        f = f + x @ W_out[t]

    e_atom = f @ w_read
    energy = jnp.sum(e_atom.reshape(B, N // B), axis=1)
    zf = z.astype(jnp.float32)
    dm = jnp.maximum(d, 1e-3)
    rep = zf[src] * zf[dst] / dm * fc
    energy = energy + 0.5 * jax.ops.segment_sum(rep, dst // (N // B), num_segments=B)

    vs = [W_out[t] @ w_read for t in range(NCONV)]
    G = jnp.broadcast_to(vs[2][None, :], (N, F))
    dBasis = jnp.zeros((E, K), dtype=jnp.float32)
    for t in range(NCONV - 1, -1, -1):
        dA = G @ W2[t].T
        dH = dA * _silu_prime(hs[t])
        dU = dH @ W1[t].T
        dUe = jnp.take(dU, dst, axis=0)
        xse = jnp.take(xs[t], src, axis=0)
        dBasis = dBasis + (dUe * xse) @ W_g[t].T
        scat = jax.ops.segment_sum(dUe * gs[t], src, num_segments=N)
        G = G + dU + scat
        if t >= 1:
            G = G + vs[t - 1][None, :]

    bprime = bern * (GAMMA * (-kk[None, :] + (K - 1.0 - kk)[None, :] * (u / c1u)[:, None]))
    ddot = jnp.sum(dBasis * (bprime * fc[:, None] + bern * fcp[:, None]), axis=1)
    drep = 0.5 * zf[src] * zf[dst] * (
        -(d > 1e-3).astype(jnp.float32) / (dm * dm) * fc + fcp / dm)
    gd = ddot + drep
    fvec = (gd / d)[:, None] * r
    forces = jax.ops.segment_sum(fvec, dst, num_segments=N) \
        - jax.ops.segment_sum(fvec, src, num_segments=N)
    return energy, forces


# speedup vs baseline: 3.0572x; 2.4840x over previous
import functools

import jax
import jax.numpy as jnp
from jax import lax
from jax.experimental import pallas as pl
from jax.experimental.pallas import tpu as pltpu
from jax.experimental.pallas import tpu_sc as plsc
from jax.scipy.special import gammaln

N = 10000
B = 100
E = 160000
F = 128
MAXZ = 87
K = 20
NCONV = 3
RCUT = 5.0
GAMMA = 0.5

_CHUNK = 128
_NW = 32


def _mesh():
    return plsc.VectorSubcoreMesh(core_axis_name="c", subcore_axis_name="s")


@functools.partial(jax.jit, static_argnames=("ncols",))
def _sc_gather(tbl, idx, ncols):
    M = idx.shape[0]
    nch = M // _CHUNK
    niter = (nch + _NW - 1) // _NW

    def body(tbl_hbm, idx_hbm, out_hbm, idx_v, rows_v, sem):
        wid = lax.axis_index("s") * 2 + lax.axis_index("c")

        def it(i, carry):
            ch = i * _NW + wid

            @pl.when(ch < nch)
            def _():
                base = ch * _CHUNK
                pltpu.sync_copy(idx_hbm.at[pl.ds(base, _CHUNK)], idx_v)
                pltpu.async_copy(tbl_hbm.at[idx_v], rows_v, sem).wait()
                pltpu.sync_copy(rows_v, out_hbm.at[pl.ds(base, _CHUNK)])
            return carry

        lax.fori_loop(0, niter, it, 0)

    return pl.kernel(
        body,
        out_type=jax.ShapeDtypeStruct((M, ncols), jnp.float32),
        mesh=_mesh(),
        scratch_types=[
            pltpu.VMEM((_CHUNK,), jnp.int32),
            pltpu.VMEM((_CHUNK, ncols), jnp.float32),
            pltpu.SemaphoreType.DMA,
        ],
        compiler_params=pltpu.CompilerParams(use_tc_tiling_on_sc=(ncols % 128 == 0)),
    )(tbl, idx)


@functools.partial(jax.jit, static_argnames=("nrows",))
def _sc_scatter_add(payload, idx, nrows):
    M, D = payload.shape
    nch = M // _CHUNK
    nch_c = nch // 2
    niter = (nch_c + 15) // 16
    nrows_pad = ((nrows + 127) // 128) * 128
    rows_sub = nrows_pad // 16
    zeros = jnp.zeros((nrows_pad, D), jnp.float32)

    def body(pay_hbm, idx_hbm, zeros_hbm, out_hbm, idx_v, rows_v, acc_sh, sem):
        cid = lax.axis_index("c")
        sid = lax.axis_index("s")
        r0 = sid * rows_sub
        pltpu.sync_copy(zeros_hbm.at[pl.ds(r0, rows_sub)],
                        acc_sh.at[pl.ds(r0, rows_sub)])
        plsc.subcore_barrier()

        def it(i, carry):
            local = i * 16 + sid

            @pl.when(local < nch_c)
            def _():
                base = (cid * nch_c + local) * _CHUNK
                pltpu.sync_copy(idx_hbm.at[pl.ds(base, _CHUNK)], idx_v)
                pltpu.sync_copy(pay_hbm.at[pl.ds(base, _CHUNK)], rows_v)
                pltpu.sync_copy(rows_v, acc_sh.at[idx_v], add=True)
            return carry

        lax.fori_loop(0, niter, it, 0)
        plsc.subcore_barrier()
        pltpu.sync_copy(acc_sh.at[pl.ds(r0, rows_sub)],
                        out_hbm.at[cid, pl.ds(r0, rows_sub)])

    out = pl.kernel(
        body,
        out_type=jax.ShapeDtypeStruct((2, nrows_pad, D), jnp.float32),
        mesh=_mesh(),
        scratch_types=[
            pltpu.VMEM((_CHUNK,), jnp.int32),
            pltpu.VMEM((_CHUNK, D), jnp.float32),
            pltpu.VMEM_SHARED((nrows_pad, D), jnp.float32),
            pltpu.SemaphoreType.DMA,
        ],
        compiler_params=pltpu.CompilerParams(use_tc_tiling_on_sc=(D % 128 == 0)),
    )(payload, idx, zeros)
    return out[:, :nrows]


def _silu(h):
    return h * jax.nn.sigmoid(h)


def _silu_prime(h):
    s = jax.nn.sigmoid(h)
    return s * (1.0 + h * (1.0 - s))


def kernel(z, xyz, nbrs, charge, spin, num_atoms, emb_z, q_vec, s_vec, W_g, W1, W2, W_out, w_read):
    mol_id = jnp.repeat(jnp.arange(B), N // B)
    src = nbrs[:, 0]
    dst = nbrs[:, 1]

    x0 = jnp.take(emb_z, z, axis=0) \
        + charge[mol_id][:, None] * q_vec[None, :] \
        + spin[mol_id][:, None] * s_vec[None, :]

    tbl = jnp.concatenate(
        [xyz, z.astype(jnp.float32)[:, None],
         jnp.zeros((N, 12), jnp.float32)], axis=1)
    tsrc = _sc_gather(tbl, src, 16)
    tdst = _sc_gather(tbl, dst, 16)
    r = tdst[:, :3] - tsrc[:, :3]
    zz = tsrc[:, 3] * tdst[:, 3]
    d = jnp.sqrt(jnp.sum(r * r, axis=-1) + 1e-12)
    valid = (src != dst).astype(jnp.float32)
    inside = (d < RCUT).astype(jnp.float32)
    fc = 0.5 * (jnp.cos(jnp.pi * d / RCUT) + 1.0) * inside * valid
    fcp = -0.5 * (jnp.pi / RCUT) * jnp.sin(jnp.pi * d / RCUT) * inside * valid

    u = jnp.exp(-GAMMA * d)
    kk = jnp.arange(K, dtype=jnp.float32)
    logbin = gammaln(float(K)) - gammaln(kk + 1.0) - gammaln(float(K) - kk)
    c1u = jnp.clip(1.0 - u, 1e-10, 1.0)
    lu = jnp.log(jnp.clip(u, 1e-10, 1.0))
    l1u = jnp.log(c1u)
    bern = jnp.exp(logbin[None, :] + kk[None, :] * lu[:, None]
                   + (K - 1.0 - kk)[None, :] * l1u[:, None])
    basis = bern * fc[:, None]

    xs = [x0]
    hs = []
    gs = []
    xes = []
    f = jnp.zeros_like(x0)
    x = x0
    for t in range(NCONV):
        g = basis @ W_g[t]
        gs.append(g)
        xe = _sc_gather(x, src, F)
        xes.append(xe)
        macc = _sc_scatter_add(xe * g, dst, N)
        m = macc[0] + macc[1]
        h = (x + m) @ W1[t]
        hs.append(h)
        x = x + _silu(h) @ W2[t]
        xs.append(x)
        f = f + x @ W_out[t]

    e_atom = f @ w_read

    vs = [W_out[t] @ w_read for t in range(NCONV)]
    G = jnp.broadcast_to(vs[2][None, :], (N, F))
    dBasis = jnp.zeros((E, K), dtype=jnp.float32)
    for t in range(NCONV - 1, -1, -1):
        dA = G @ W2[t].T
        dH = dA * _silu_prime(hs[t])
        dU = dH @ W1[t].T
        dUe = _sc_gather(dU, dst, F)
        dBasis = dBasis + (dUe * xes[t]) @ W_g[t].T
        if t >= 1:
            sacc = _sc_scatter_add(dUe * gs[t], src, N)
            G = G + dU + sacc[0] + sacc[1] + vs[t - 1][None, :]

    bprime = bern * (GAMMA * (-kk[None, :]
                              + (K - 1.0 - kk)[None, :] * (u / c1u)[:, None]))
    ddot = jnp.sum(dBasis * (bprime * fc[:, None] + bern * fcp[:, None]), axis=1)
    dm = jnp.maximum(d, 1e-3)
    drep = 0.5 * zz * (-(d > 1e-3).astype(jnp.float32) / (dm * dm) * fc
                       + fcp / dm)
    gd = ddot + drep
    fvec = (gd / d)[:, None] * r
    rep = zz / dm * fc

    pay = jnp.zeros((2 * E, 16), jnp.float32)
    pay = pay.at[:E, :3].set(fvec).at[:E, 3].set(rep)
    pay = pay.at[E:, :3].set(-fvec)
    facc = _sc_scatter_add(pay, jnp.concatenate([dst, src]), N)
    fsum = facc[0] + facc[1]
    forces = fsum[:, :3]
    rep_atom = fsum[:, 3]

    energy = jnp.sum((e_atom + 0.5 * rep_atom).reshape(B, N // B), axis=1)
    return energy, forces


# speedup vs baseline: 3.0803x; 1.0076x over previous
import functools

import jax
import jax.numpy as jnp
import numpy as np
from jax import lax
from jax.experimental import pallas as pl
from jax.experimental.pallas import tpu as pltpu
from jax.experimental.pallas import tpu_sc as plsc

N = 10000
B = 100
E = 160000
F = 128
MAXZ = 87
K = 20
NCONV = 3
RCUT = 5.0
GAMMA = 0.5

_CHUNK = 128
_NW = 32
_BE = 1280
_NEB = E // _BE
_BN = 2000

from math import lgamma as _lgamma

_LOGBIN = [_lgamma(K) - _lgamma(k + 1.0) - _lgamma(K - k) for k in range(K)]


def _f32(x):
    return x.astype(jnp.float32)


def _mesh():
    return plsc.VectorSubcoreMesh(core_axis_name="c", subcore_axis_name="s")


@functools.partial(jax.jit, static_argnames=("ncols",))
def _sc_gather(tbl, idx, ncols):
    M = idx.shape[0]
    nch = M // _CHUNK
    niter = (nch + _NW - 1) // _NW

    def body(tbl_hbm, idx_hbm, out_hbm, idx_v, rows_v, sem):
        wid = lax.axis_index("s") * 2 + lax.axis_index("c")

        def it(i, carry):
            ch = i * _NW + wid

            @pl.when(ch < nch)
            def _():
                base = ch * _CHUNK
                pltpu.sync_copy(idx_hbm.at[pl.ds(base, _CHUNK)], idx_v)
                pltpu.async_copy(tbl_hbm.at[idx_v], rows_v, sem).wait()
                pltpu.sync_copy(rows_v, out_hbm.at[pl.ds(base, _CHUNK)])
            return carry

        lax.fori_loop(0, niter, it, 0)

    return pl.kernel(
        body,
        out_type=jax.ShapeDtypeStruct((M, ncols), jnp.float32),
        mesh=_mesh(),
        scratch_types=[
            pltpu.VMEM((_CHUNK,), jnp.int32),
            pltpu.VMEM((_CHUNK, ncols), jnp.float32),
            pltpu.SemaphoreType.DMA,
        ],
        compiler_params=pltpu.CompilerParams(use_tc_tiling_on_sc=(ncols % 128 == 0)),
    )(tbl, idx)


@functools.partial(jax.jit, static_argnames=("nrows",))
def _sc_scatter_add(payload, idx, nrows):
    M, D = payload.shape
    nch = M // _CHUNK
    nch_c = nch // 2
    niter = (nch_c + 15) // 16
    nrows_pad = ((nrows + 127) // 128) * 128
    rows_sub = nrows_pad // 16
    zeros = jnp.zeros((nrows_pad, D), jnp.float32)

    def body(pay_hbm, idx_hbm, zeros_hbm, out_hbm, idx_v, rows_v, acc_sh, sem):
        cid = lax.axis_index("c")
        sid = lax.axis_index("s")
        r0 = sid * rows_sub
        pltpu.sync_copy(zeros_hbm.at[pl.ds(r0, rows_sub)],
                        acc_sh.at[pl.ds(r0, rows_sub)])
        plsc.subcore_barrier()

        def it(i, carry):
            local = i * 16 + sid

            @pl.when(local < nch_c)
            def _():
                base = (cid * nch_c + local) * _CHUNK
                pltpu.sync_copy(idx_hbm.at[pl.ds(base, _CHUNK)], idx_v)
                pltpu.sync_copy(pay_hbm.at[pl.ds(base, _CHUNK)], rows_v)
                pltpu.sync_copy(rows_v, acc_sh.at[idx_v], add=True)
            return carry

        lax.fori_loop(0, niter, it, 0)
        plsc.subcore_barrier()
        pltpu.sync_copy(acc_sh.at[pl.ds(r0, rows_sub)],
                        out_hbm.at[cid, pl.ds(r0, rows_sub)])

    out = pl.kernel(
        body,
        out_type=jax.ShapeDtypeStruct((2, nrows_pad, D), jnp.float32),
        mesh=_mesh(),
        scratch_types=[
            pltpu.VMEM((_CHUNK,), jnp.int32),
            pltpu.VMEM((_CHUNK, D), jnp.float32),
            pltpu.VMEM_SHARED((nrows_pad, D), jnp.float32),
            pltpu.SemaphoreType.DMA,
        ],
        compiler_params=pltpu.CompilerParams(use_tc_tiling_on_sc=(D % 128 == 0)),
    )(payload, idx, zeros)
    return out[:, :nrows]


def _dot(a, b, dims):
    return lax.dot_general(a, b, (dims, ((), ())),
                           preferred_element_type=jnp.float32)


def _silu(h):
    return h * jax.nn.sigmoid(h)


def _silu_prime(h):
    s = jax.nn.sigmoid(h)
    return s * (1.0 + h * (1.0 - s))


def _edge_geom(tsrc, tdst, validf):
    def body(ts_ref, td_ref, va_ref, geo_ref, bas_ref):
        ts = ts_ref[...]
        td = td_ref[...]
        va = va_ref[...][0, 0]
        lane = lax.broadcasted_iota(jnp.int32, (1, 16), 1)
        m3 = (lane < 3).astype(jnp.float32)
        e3 = (lane == 3).astype(jnp.float32)
        dr = (td - ts) * m3
        d = jnp.sqrt(jnp.sum(dr * dr, axis=1) + 1e-12)
        zz = jnp.sum(ts * e3, axis=1) * jnp.sum(td * e3, axis=1)
        inside = (d < RCUT).astype(jnp.float32) * va
        fc = 0.5 * (jnp.cos(jnp.pi * d / RCUT) + 1.0) * inside
        fcp = -0.5 * (jnp.pi / RCUT) * jnp.sin(jnp.pi * d / RCUT) * inside
        u = jnp.exp(-GAMMA * d)
        c1u = jnp.clip(1.0 - u, 1e-10, 1.0)
        lu = jnp.log(jnp.clip(u, 1e-10, 1.0))
        l1u = jnp.log(c1u)
        urat = u / c1u
        zero = jnp.zeros_like(d)
        geo_ref[...] = jnp.concatenate(
            [v[None, :] for v in (d, fc, fcp, lu, l1u, urat, zz, zero)], axis=0)
        rows = [jnp.exp(_LOGBIN[k] + k * lu + (K - 1.0 - k) * l1u) * fc
                for k in range(K)] + [zero] * (32 - K)
        bas_ref[...] = jnp.concatenate([v[None, :] for v in rows], axis=0)

    return pl.pallas_call(
        body,
        grid=(_NEB,),
        in_specs=[pl.BlockSpec((_BE, 16), lambda i: (i, 0)),
                  pl.BlockSpec((_BE, 16), lambda i: (i, 0)),
                  pl.BlockSpec((1, 1, _BE), lambda i: (i, 0, 0))],
        out_specs=[pl.BlockSpec((8, _BE), lambda i: (0, i)),
                   pl.BlockSpec((32, _BE), lambda i: (0, i))],
        out_shape=[jax.ShapeDtypeStruct((8, E), jnp.float32),
                   jax.ShapeDtypeStruct((32, E), jnp.float32)],
    )(tsrc, tdst, validf.reshape(_NEB, 1, _BE))


def _msg(basT, xe, wg):
    def body(bas_ref, xe_ref, wg_ref, out_ref):
        g = _dot(bas_ref[...], wg_ref[...], ((0,), (0,)))
        out_ref[...] = xe_ref[...] * g

    return pl.pallas_call(
        body,
        grid=(_NEB,),
        in_specs=[pl.BlockSpec((32, _BE), lambda i: (0, i)),
                  pl.BlockSpec((_BE, F), lambda i: (i, 0)),
                  pl.BlockSpec((32, F), lambda i: (0, 0))],
        out_specs=pl.BlockSpec((_BE, F), lambda i: (i, 0)),
        out_shape=jax.ShapeDtypeStruct((E, F), jnp.float32),
    )(basT, xe, wg)


def _node(x, macc, f_in, w1, w2, wo):
    def body(x_ref, m_ref, f_ref, w1_ref, w2_ref, wo_ref,
             xn_ref, h_ref, fo_ref):
        x = x_ref[...]
        m = m_ref[0] + m_ref[1]
        h = _dot(x + m, w1_ref[...], ((1,), (0,)))
        xn = x + _dot(_silu(h), w2_ref[...], ((1,), (0,)))
        xn_ref[...] = xn
        h_ref[...] = h
        fo_ref[...] = f_ref[...] + _dot(xn, wo_ref[...], ((1,), (0,)))

    nb = N // _BN
    return pl.pallas_call(
        body,
        grid=(nb,),
        in_specs=[pl.BlockSpec((_BN, F), lambda i: (i, 0)),
                  pl.BlockSpec((2, _BN, F), lambda i: (0, i, 0)),
                  pl.BlockSpec((_BN, F), lambda i: (i, 0)),
                  pl.BlockSpec((F, F), lambda i: (0, 0)),
                  pl.BlockSpec((F, F), lambda i: (0, 0)),
                  pl.BlockSpec((F, F), lambda i: (0, 0))],
        out_specs=[pl.BlockSpec((_BN, F), lambda i: (i, 0)),
                   pl.BlockSpec((_BN, F), lambda i: (i, 0)),
                   pl.BlockSpec((_BN, F), lambda i: (i, 0))],
        out_shape=[jax.ShapeDtypeStruct((N, F), jnp.float32)] * 3,
    )(x, macc, f_in, w1, w2, wo)


def _bwd_node(h, w1, w2, wv, wread, dU_prev=None, scat=None, g_prev=None,
              out_g=False):
    have_du = dU_prev is not None
    have_g = g_prev is not None

    def body(*refs):
        it = iter(refs)
        h_ref = next(it)
        w1_ref = next(it)
        w2_ref = next(it)
        wv_ref = next(it)
        wr_ref = next(it)
        du_ref = next(it) if have_du else None
        sc_ref = next(it) if have_du else None
        gp_ref = next(it) if have_g else None
        outs = list(it)
        v = _dot(wr_ref[...], wv_ref[...], ((1,), (1,)))
        g = jnp.broadcast_to(v, (_BN, F))
        if have_du:
            g = g + du_ref[...] + sc_ref[0] + sc_ref[1]
        if have_g:
            g = g + gp_ref[...]
        dA = _dot(g, w2_ref[...], ((1,), (1,)))
        dU = _dot(dA * _silu_prime(h_ref[...]), w1_ref[...], ((1,), (1,)))
        if out_g:
            outs[0][...] = g
            outs[1][...] = dU
        else:
            outs[0][...] = dU

    nb = N // _BN
    nf = pl.BlockSpec((_BN, F), lambda i: (i, 0))
    ff = pl.BlockSpec((F, F), lambda i: (0, 0))
    in_specs = [nf, ff, ff, ff, pl.BlockSpec((1, F), lambda i: (0, 0))]
    args = [h, w1, w2, wv, wread.reshape(1, F)]
    if have_du:
        in_specs += [nf, pl.BlockSpec((2, _BN, F), lambda i: (0, i, 0))]
        args += [dU_prev, scat]
    if have_g:
        in_specs += [nf]
        args += [g_prev]
    nout = 2 if out_g else 1
    out = pl.pallas_call(
        body,
        grid=(nb,),
        in_specs=in_specs,
        out_specs=[nf] * nout,
        out_shape=[jax.ShapeDtypeStruct((N, F), jnp.float32)] * nout,
    )(*args)
    return out if out_g else (None, out[0])


def _bwd_edge(dUe, xe, basT, wg, dBasT, with_pay):
    def body(du_ref, xe_ref, bas_ref, wg_ref, dbin_ref, dbout_ref, *pay_ref):
        du = du_ref[...]
        q = du * xe_ref[...]
        dbout_ref[...] = dbin_ref[...] + _dot(wg_ref[...], q, ((1,), (1,)))
        if with_pay:
            g = _dot(bas_ref[...], wg_ref[...], ((0,), (0,)))
            pay_ref[0][...] = du * g

    ef = pl.BlockSpec((_BE, F), lambda i: (i, 0))
    bs = pl.BlockSpec((32, _BE), lambda i: (0, i))
    outs = [jax.ShapeDtypeStruct((32, E), jnp.float32)]
    out_specs = [bs]
    if with_pay:
        outs.append(jax.ShapeDtypeStruct((E, F), jnp.float32))
        out_specs.append(ef)
    return pl.pallas_call(
        body,
        grid=(_NEB,),
        in_specs=[ef, ef, bs, pl.BlockSpec((32, F), lambda i: (0, 0)), bs],
        out_specs=out_specs,
        out_shape=outs,
        input_output_aliases={4: 0},
    )(dUe, xe, basT, wg, dBasT)


def _force_pay(tsrc, tdst, geoT, dBasT):
    def body(ts_ref, td_ref, geo_ref, db_ref, out_ref):
        geo = geo_ref[...]
        db = db_ref[...]
        d, fc, fcp = geo[0], geo[1], geo[2]
        lu, l1u, urat, zz = geo[3], geo[4], geo[5], geo[6]
        acc = jnp.zeros_like(d)
        for k in range(K):
            bern = jnp.exp(_LOGBIN[k] + k * lu + (K - 1.0 - k) * l1u)
            bp = bern * (GAMMA * (-float(k) + (K - 1.0 - k) * urat))
            acc = acc + db[k] * (bp * fc + bern * fcp)
        dm = jnp.maximum(d, 1e-3)
        drep = 0.5 * zz * (-(d > 1e-3).astype(jnp.float32) / (dm * dm) * fc
                           + fcp / dm)
        coef = (acc + drep) / d
        rep = zz / dm * fc
        lane = lax.broadcasted_iota(jnp.int32, (1, 16), 1)
        m3 = (lane < 3).astype(jnp.float32)
        e3 = (lane == 3).astype(jnp.float32)
        dr = (td_ref[...] - ts_ref[...]) * m3
        fv = dr * coef[:, None]
        out_ref[0] = fv + rep[:, None] * e3
        out_ref[1] = -fv

    return pl.pallas_call(
        body,
        grid=(_NEB,),
        in_specs=[pl.BlockSpec((_BE, 16), lambda i: (i, 0)),
                  pl.BlockSpec((_BE, 16), lambda i: (i, 0)),
                  pl.BlockSpec((8, _BE), lambda i: (0, i)),
                  pl.BlockSpec((32, _BE), lambda i: (0, i))],
        out_specs=pl.BlockSpec((2, _BE, 16), lambda i: (0, i, 0)),
        out_shape=jax.ShapeDtypeStruct((2, E, 16), jnp.float32),
    )(tsrc, tdst, geoT, dBasT)


def _embed(emb_rows, charge, spin, q_vec, s_vec):
    bn = 1000
    nb = N // bn

    def body(e_ref, c_ref, s_ref, q_ref, sv_ref, out_ref):
        cs = c_ref[...][:, None] * q_ref[...][None, :] \
            + s_ref[...][:, None] * sv_ref[...][None, :]
        i = pl.program_id(0)
        jj = (lax.broadcasted_iota(jnp.int32, (bn, B), 0) + i * bn) // (N // B)
        ii = lax.broadcasted_iota(jnp.int32, (bn, B), 1)
        sel = (jj == ii).astype(jnp.float32)
        out_ref[...] = e_ref[...] + _dot(sel, cs, ((1,), (0,)))

    return pl.pallas_call(
        body,
        grid=(nb,),
        in_specs=[pl.BlockSpec((bn, F), lambda i: (i, 0)),
                  pl.BlockSpec((B,), lambda i: (0,)),
                  pl.BlockSpec((B,), lambda i: (0,)),
                  pl.BlockSpec((F,), lambda i: (0,)),
                  pl.BlockSpec((F,), lambda i: (0,))],
        out_specs=pl.BlockSpec((bn, F), lambda i: (i, 0)),
        out_shape=jax.ShapeDtypeStruct((N, F), jnp.float32),
    )(emb_rows, charge, spin, q_vec, s_vec)


def _readout(f, rep_atom, wread):
    def body(f_ref, r_ref, w_ref, out_ref):
        e = _dot(f_ref[...], w_ref[...], ((1,), (1,)))[:, 0] + 0.5 * r_ref[...]
        ii = lax.broadcasted_iota(jnp.int32, (B, N), 0)
        jj = lax.broadcasted_iota(jnp.int32, (B, N), 1) // (N // B)
        sel = (ii == jj).astype(jnp.float32)
        out_ref[...] = jnp.sum(sel * e[None, :], axis=1)

    return pl.pallas_call(
        body,
        grid=(1,),
        in_specs=[pl.BlockSpec((N, F), lambda i: (0, 0)),
                  pl.BlockSpec((N,), lambda i: (0,)),
                  pl.BlockSpec((1, F), lambda i: (0, 0))],
        out_specs=pl.BlockSpec((B,), lambda i: (0,)),
        out_shape=jax.ShapeDtypeStruct((B,), jnp.float32),
    )(f, rep_atom, wread.reshape(1, F))


def kernel(z, xyz, nbrs, charge, spin, num_atoms, emb_z, q_vec, s_vec, W_g, W1, W2, W_out, w_read):
    src = nbrs[:, 0]
    dst = nbrs[:, 1]
    validf = (src != dst).astype(jnp.float32)

    zpad = jnp.concatenate([z, jnp.zeros((128 - N % 128,), z.dtype)]) \
        if N % 128 else z
    emb_rows = _sc_gather(emb_z, zpad, F)[:N]
    x0 = _embed(emb_rows, charge, spin, q_vec, s_vec)

    tbl = jnp.concatenate(
        [xyz, _f32(z)[:, None], jnp.zeros((N, 12), jnp.float32)], axis=1)
    tsrc = _sc_gather(tbl, src, 16)
    tdst = _sc_gather(tbl, dst, 16)
    geoT, basT = _edge_geom(tsrc, tdst, validf)

    wg_pad = [jnp.concatenate([W_g[t], jnp.zeros((32 - K, F), jnp.float32)],
                              axis=0) for t in range(NCONV)]

    hs = []
    xes = []
    f = jnp.zeros((N, F), jnp.float32)
    x = x0
    for t in range(NCONV):
        xe = _sc_gather(x, src, F)
        xes.append(xe)
        msg = _msg(basT, xe, wg_pad[t])
        macc = _sc_scatter_add(msg, dst, N)
        x, h, f = _node(x, macc, f, W1[t], W2[t], W_out[t])
        hs.append(h)

    dBasT = jnp.zeros((32, E), jnp.float32)
    _, dU = _bwd_node(hs[2], W1[2], W2[2], W_out[2], w_read)
    dUe = _sc_gather(dU, dst, F)
    dBasT, pay = _bwd_edge(dUe, xes[2], basT, wg_pad[2], dBasT, True)
    scat = _sc_scatter_add(pay, src, N)

    g2, dU1 = _bwd_node(hs[1], W1[1], W2[1], W_out[1] + W_out[2], w_read,
                        dU_prev=dU, scat=scat, out_g=True)
    dUe = _sc_gather(dU1, dst, F)
    dBasT, pay = _bwd_edge(dUe, xes[1], basT, wg_pad[1], dBasT, True)
    scat = _sc_scatter_add(pay, src, N)

    _, dU0 = _bwd_node(hs[0], W1[0], W2[0], W_out[0], w_read,
                       dU_prev=dU1, scat=scat, g_prev=g2)
    dUe = _sc_gather(dU0, dst, F)
    dBasT = _bwd_edge(dUe, xes[0], basT, wg_pad[0], dBasT, False)[0]

    pay2 = _force_pay(tsrc, tdst, geoT, dBasT).reshape(2 * E, 16)
    facc = _sc_scatter_add(pay2, jnp.concatenate([dst, src]), N)
    fsum = facc[0] + facc[1]
    forces = fsum[:, :3]

    energy = _readout(f, fsum[:, 3], w_read)
    return energy, forces


# speedup vs baseline: 3.5354x; 1.1477x over previous
import functools

import jax
import jax.numpy as jnp
import numpy as np
from jax import lax
from jax.experimental import pallas as pl
from jax.experimental.pallas import tpu as pltpu
from jax.experimental.pallas import tpu_sc as plsc

N = 10000
B = 100
E = 160000
F = 128
MAXZ = 87
K = 20
NCONV = 3
RCUT = 5.0
GAMMA = 0.5

_CHUNK = 128
_NW = 32
_BE = 1280
_NEB = E // _BE
_BN = 2000

from math import lgamma as _lgamma

_LOGBIN = [_lgamma(K) - _lgamma(k + 1.0) - _lgamma(K - k) for k in range(K)]


def _f32(x):
    return x.astype(jnp.float32)


def _mesh():
    return plsc.VectorSubcoreMesh(core_axis_name="c", subcore_axis_name="s")


_GRP = 5


@functools.partial(jax.jit, static_argnames=("ncols",))
def _sc_gather(tbl, idx, ncols):
    M = idx.shape[0]
    nch = M // _CHUNK
    ngr = nch // _GRP
    gpw = (ngr + _NW - 1) // _NW
    grows = _GRP * _CHUNK
    idx2 = idx.reshape(nch, _CHUNK)

    def body(tbl_hbm, idx_hbm, out_hbm, idx_v, rows_v, semi, semg):
        wid = lax.axis_index("s") * 2 + lax.axis_index("c")

        def it(i, carry):
            gr = i * _NW + wid

            @pl.when(gr < ngr)
            def _():
                di = [pltpu.async_copy(idx_hbm.at[gr * _GRP + j],
                                       idx_v.at[j], semi)
                      for j in range(_GRP)]
                for dsc in di:
                    dsc.wait()
                descs = [pltpu.async_copy(
                    tbl_hbm.at[idx_v.at[j]],
                    rows_v.at[pl.ds(j * _CHUNK, _CHUNK)], semg)
                    for j in range(_GRP)]
                for dsc in descs:
                    dsc.wait()
                pltpu.sync_copy(rows_v, out_hbm.at[pl.ds(gr * grows, grows)])
            return carry

        lax.fori_loop(0, gpw, it, 0)

    return pl.kernel(
        body,
        out_type=jax.ShapeDtypeStruct((M, ncols), jnp.float32),
        mesh=_mesh(),
        scratch_types=[
            pltpu.VMEM((_GRP, _CHUNK), jnp.int32),
            pltpu.VMEM((grows, ncols), jnp.float32),
            pltpu.SemaphoreType.DMA,
            pltpu.SemaphoreType.DMA,
        ],
        compiler_params=pltpu.CompilerParams(use_tc_tiling_on_sc=(ncols % 128 == 0)),
    )(tbl, idx2)


@functools.partial(jax.jit, static_argnames=("nrows",))
def _sc_scatter_add(payload, idx, nrows):
    M, D = payload.shape
    grp = _GRP if D <= 32 else 2
    nch = M // _CHUNK
    ngr = nch // grp
    half = (ngr + 1) // 2
    niter = (half + 15) // 16
    grows = grp * _CHUNK
    nrows_pad = ((nrows + 127) // 128) * 128
    rows_sub = nrows_pad // 16
    zeros = jnp.zeros((nrows_pad, D), jnp.float32)
    idx2 = idx.reshape(nch, _CHUNK)

    def body(pay_hbm, idx_hbm, zeros_hbm, out_hbm, idx_v, rows_v, acc_sh,
             semi, semp, sema):
        cid = lax.axis_index("c")
        sid = lax.axis_index("s")
        r0 = sid * rows_sub
        pltpu.sync_copy(zeros_hbm.at[pl.ds(r0, rows_sub)],
                        acc_sh.at[pl.ds(r0, rows_sub)])
        plsc.subcore_barrier()

        def it(i, carry):
            gr = cid * half + i * 16 + sid

            @pl.when((i * 16 + sid < half) & (gr < ngr))
            def _():
                di = [pltpu.async_copy(idx_hbm.at[gr * grp + j],
                                       idx_v.at[j], semi)
                      for j in range(grp)]
                dp = pltpu.async_copy(pay_hbm.at[pl.ds(gr * grows, grows)],
                                      rows_v, semp)
                for dsc in di:
                    dsc.wait()
                dp.wait()
                descs = [pltpu.async_copy(
                    rows_v.at[pl.ds(j * _CHUNK, _CHUNK)],
                    acc_sh.at[idx_v.at[j]], sema, add=True)
                    for j in range(grp)]
                for dsc in descs:
                    dsc.wait()
            return carry

        lax.fori_loop(0, niter, it, 0)
        plsc.subcore_barrier()
        pltpu.sync_copy(acc_sh.at[pl.ds(r0, rows_sub)],
                        out_hbm.at[cid, pl.ds(r0, rows_sub)])

    out = pl.kernel(
        body,
        out_type=jax.ShapeDtypeStruct((2, nrows_pad, D), jnp.float32),
        mesh=_mesh(),
        scratch_types=[
            pltpu.VMEM((grp, _CHUNK), jnp.int32),
            pltpu.VMEM((grows, D), jnp.float32),
            pltpu.VMEM_SHARED((nrows_pad, D), jnp.float32),
            pltpu.SemaphoreType.DMA,
            pltpu.SemaphoreType.DMA,
            pltpu.SemaphoreType.DMA,
        ],
        compiler_params=pltpu.CompilerParams(use_tc_tiling_on_sc=(D % 128 == 0)),
    )(payload, idx2, zeros)
    return out[:, :nrows]


def _dot(a, b, dims):
    return lax.dot_general(a, b, (dims, ((), ())),
                           preferred_element_type=jnp.float32)


def _silu(h):
    return h * jax.nn.sigmoid(h)


def _silu_prime(h):
    s = jax.nn.sigmoid(h)
    return s * (1.0 + h * (1.0 - s))


def _edge_geom(tsrc, tdst, validf):
    def body(ts_ref, td_ref, va_ref, geo_ref, bas_ref):
        ts = ts_ref[...]
        td = td_ref[...]
        va = va_ref[...][0, 0]
        lane = lax.broadcasted_iota(jnp.int32, (1, 16), 1)
        m3 = (lane < 3).astype(jnp.float32)
        e3 = (lane == 3).astype(jnp.float32)
        dr = (td - ts) * m3
        d = jnp.sqrt(jnp.sum(dr * dr, axis=1) + 1e-12)
        zz = jnp.sum(ts * e3, axis=1) * jnp.sum(td * e3, axis=1)
        inside = (d < RCUT).astype(jnp.float32) * va
        fc = 0.5 * (jnp.cos(jnp.pi * d / RCUT) + 1.0) * inside
        fcp = -0.5 * (jnp.pi / RCUT) * jnp.sin(jnp.pi * d / RCUT) * inside
        u = jnp.exp(-GAMMA * d)
        c1u = jnp.clip(1.0 - u, 1e-10, 1.0)
        lu = jnp.log(jnp.clip(u, 1e-10, 1.0))
        l1u = jnp.log(c1u)
        urat = u / c1u
        zero = jnp.zeros_like(d)
        geo_ref[...] = jnp.concatenate(
            [v[None, :] for v in (d, fc, fcp, lu, l1u, urat, zz, zero)], axis=0)
        rows = [jnp.exp(_LOGBIN[k] + k * lu + (K - 1.0 - k) * l1u) * fc
                for k in range(K)] + [zero] * (32 - K)
        bas_ref[...] = jnp.concatenate([v[None, :] for v in rows], axis=0)

    return pl.pallas_call(
        body,
        grid=(_NEB,),
        in_specs=[pl.BlockSpec((_BE, 16), lambda i: (i, 0)),
                  pl.BlockSpec((_BE, 16), lambda i: (i, 0)),
                  pl.BlockSpec((1, 1, _BE), lambda i: (i, 0, 0))],
        out_specs=[pl.BlockSpec((8, _BE), lambda i: (0, i)),
                   pl.BlockSpec((32, _BE), lambda i: (0, i))],
        out_shape=[jax.ShapeDtypeStruct((8, E), jnp.float32),
                   jax.ShapeDtypeStruct((32, E), jnp.float32)],
    )(tsrc, tdst, validf.reshape(_NEB, 1, _BE))


def _msg(basT, xe, wg):
    def body(bas_ref, xe_ref, wg_ref, out_ref):
        g = _dot(bas_ref[...], wg_ref[...], ((0,), (0,)))
        out_ref[...] = xe_ref[...] * g

    return pl.pallas_call(
        body,
        grid=(_NEB,),
        in_specs=[pl.BlockSpec((32, _BE), lambda i: (0, i)),
                  pl.BlockSpec((_BE, F), lambda i: (i, 0)),
                  pl.BlockSpec((32, F), lambda i: (0, 0))],
        out_specs=pl.BlockSpec((_BE, F), lambda i: (i, 0)),
        out_shape=jax.ShapeDtypeStruct((E, F), jnp.float32),
    )(basT, xe, wg)


def _node(x, macc, f_in, w1, w2, wo):
    def body(x_ref, m_ref, f_ref, w1_ref, w2_ref, wo_ref,
             xn_ref, h_ref, fo_ref):
        x = x_ref[...]
        m = m_ref[0] + m_ref[1]
        h = _dot(x + m, w1_ref[...], ((1,), (0,)))
        xn = x + _dot(_silu(h), w2_ref[...], ((1,), (0,)))
        xn_ref[...] = xn
        h_ref[...] = h
        fo_ref[...] = f_ref[...] + _dot(xn, wo_ref[...], ((1,), (0,)))

    nb = N // _BN
    return pl.pallas_call(
        body,
        grid=(nb,),
        in_specs=[pl.BlockSpec((_BN, F), lambda i: (i, 0)),
                  pl.BlockSpec((2, _BN, F), lambda i: (0, i, 0)),
                  pl.BlockSpec((_BN, F), lambda i: (i, 0)),
                  pl.BlockSpec((F, F), lambda i: (0, 0)),
                  pl.BlockSpec((F, F), lambda i: (0, 0)),
                  pl.BlockSpec((F, F), lambda i: (0, 0))],
        out_specs=[pl.BlockSpec((_BN, F), lambda i: (i, 0)),
                   pl.BlockSpec((_BN, F), lambda i: (i, 0)),
                   pl.BlockSpec((_BN, F), lambda i: (i, 0))],
        out_shape=[jax.ShapeDtypeStruct((N, F), jnp.float32)] * 3,
    )(x, macc, f_in, w1, w2, wo)


def _bwd_node(h, w1, w2, wv, wread, dU_prev=None, scat=None, g_prev=None,
              out_g=False):
    have_du = dU_prev is not None
    have_g = g_prev is not None

    def body(*refs):
        it = iter(refs)
        h_ref = next(it)
        w1_ref = next(it)
        w2_ref = next(it)
        wv_ref = next(it)
        wr_ref = next(it)
        du_ref = next(it) if have_du else None
        sc_ref = next(it) if have_du else None
        gp_ref = next(it) if have_g else None
        outs = list(it)
        v = _dot(wr_ref[...], wv_ref[...], ((1,), (1,)))
        g = jnp.broadcast_to(v, (_BN, F))
        if have_du:
            g = g + du_ref[...] + sc_ref[0] + sc_ref[1]
        if have_g:
            g = g + gp_ref[...]
        dA = _dot(g, w2_ref[...], ((1,), (1,)))
        dU = _dot(dA * _silu_prime(h_ref[...]), w1_ref[...], ((1,), (1,)))
        if out_g:
            outs[0][...] = g
            outs[1][...] = dU
        else:
            outs[0][...] = dU

    nb = N // _BN
    nf = pl.BlockSpec((_BN, F), lambda i: (i, 0))
    ff = pl.BlockSpec((F, F), lambda i: (0, 0))
    in_specs = [nf, ff, ff, ff, pl.BlockSpec((1, F), lambda i: (0, 0))]
    args = [h, w1, w2, wv, wread.reshape(1, F)]
    if have_du:
        in_specs += [nf, pl.BlockSpec((2, _BN, F), lambda i: (0, i, 0))]
        args += [dU_prev, scat]
    if have_g:
        in_specs += [nf]
        args += [g_prev]
    nout = 2 if out_g else 1
    out = pl.pallas_call(
        body,
        grid=(nb,),
        in_specs=in_specs,
        out_specs=[nf] * nout,
        out_shape=[jax.ShapeDtypeStruct((N, F), jnp.float32)] * nout,
    )(*args)
    return out if out_g else (None, out[0])


def _bwd_edge(dUe, xe, basT, wg, dBasT, with_pay):
    def body(du_ref, xe_ref, bas_ref, wg_ref, dbin_ref, dbout_ref, *pay_ref):
        du = du_ref[...]
        q = du * xe_ref[...]
        dbout_ref[...] = dbin_ref[...] + _dot(wg_ref[...], q, ((1,), (1,)))
        if with_pay:
            g = _dot(bas_ref[...], wg_ref[...], ((0,), (0,)))
            pay_ref[0][...] = du * g

    ef = pl.BlockSpec((_BE, F), lambda i: (i, 0))
    bs = pl.BlockSpec((32, _BE), lambda i: (0, i))
    outs = [jax.ShapeDtypeStruct((32, E), jnp.float32)]
    out_specs = [bs]
    if with_pay:
        outs.append(jax.ShapeDtypeStruct((E, F), jnp.float32))
        out_specs.append(ef)
    return pl.pallas_call(
        body,
        grid=(_NEB,),
        in_specs=[ef, ef, bs, pl.BlockSpec((32, F), lambda i: (0, 0)), bs],
        out_specs=out_specs,
        out_shape=outs,
        input_output_aliases={4: 0},
    )(dUe, xe, basT, wg, dBasT)


def _force_pay(tsrc, tdst, geoT, dBasT):
    def body(ts_ref, td_ref, geo_ref, db_ref, out_ref):
        geo = geo_ref[...]
        db = db_ref[...]
        d, fc, fcp = geo[0], geo[1], geo[2]
        lu, l1u, urat, zz = geo[3], geo[4], geo[5], geo[6]
        acc = jnp.zeros_like(d)
        for k in range(K):
            bern = jnp.exp(_LOGBIN[k] + k * lu + (K - 1.0 - k) * l1u)
            bp = bern * (GAMMA * (-float(k) + (K - 1.0 - k) * urat))
            acc = acc + db[k] * (bp * fc + bern * fcp)
        dm = jnp.maximum(d, 1e-3)
        drep = 0.5 * zz * (-(d > 1e-3).astype(jnp.float32) / (dm * dm) * fc
                           + fcp / dm)
        coef = (acc + drep) / d
        rep = zz / dm * fc
        lane = lax.broadcasted_iota(jnp.int32, (1, 16), 1)
        m3 = (lane < 3).astype(jnp.float32)
        e3 = (lane == 3).astype(jnp.float32)
        dr = (td_ref[...] - ts_ref[...]) * m3
        fv = dr * coef[:, None]
        out_ref[0] = fv + rep[:, None] * e3
        out_ref[1] = -fv

    return pl.pallas_call(
        body,
        grid=(_NEB,),
        in_specs=[pl.BlockSpec((_BE, 16), lambda i: (i, 0)),
                  pl.BlockSpec((_BE, 16), lambda i: (i, 0)),
                  pl.BlockSpec((8, _BE), lambda i: (0, i)),
                  pl.BlockSpec((32, _BE), lambda i: (0, i))],
        out_specs=pl.BlockSpec((2, _BE, 16), lambda i: (0, i, 0)),
        out_shape=jax.ShapeDtypeStruct((2, E, 16), jnp.float32),
    )(tsrc, tdst, geoT, dBasT)


def _embed(emb_rows, charge, spin, q_vec, s_vec):
    bn = 1000
    nb = N // bn

    def body(e_ref, c_ref, s_ref, q_ref, sv_ref, out_ref):
        cs = c_ref[...][:, None] * q_ref[...][None, :] \
            + s_ref[...][:, None] * sv_ref[...][None, :]
        i = pl.program_id(0)
        jj = (lax.broadcasted_iota(jnp.int32, (bn, B), 0) + i * bn) // (N // B)
        ii = lax.broadcasted_iota(jnp.int32, (bn, B), 1)
        sel = (jj == ii).astype(jnp.float32)
        out_ref[...] = e_ref[...] + _dot(sel, cs, ((1,), (0,)))

    return pl.pallas_call(
        body,
        grid=(nb,),
        in_specs=[pl.BlockSpec((bn, F), lambda i: (i, 0)),
                  pl.BlockSpec((B,), lambda i: (0,)),
                  pl.BlockSpec((B,), lambda i: (0,)),
                  pl.BlockSpec((F,), lambda i: (0,)),
                  pl.BlockSpec((F,), lambda i: (0,))],
        out_specs=pl.BlockSpec((bn, F), lambda i: (i, 0)),
        out_shape=jax.ShapeDtypeStruct((N, F), jnp.float32),
    )(emb_rows, charge, spin, q_vec, s_vec)


def _readout(f, rep_atom, wread):
    def body(f_ref, r_ref, w_ref, out_ref):
        e = _dot(f_ref[...], w_ref[...], ((1,), (1,)))[:, 0] + 0.5 * r_ref[...]
        ii = lax.broadcasted_iota(jnp.int32, (B, N), 0)
        jj = lax.broadcasted_iota(jnp.int32, (B, N), 1) // (N // B)
        sel = (ii == jj).astype(jnp.float32)
        out_ref[...] = jnp.sum(sel * e[None, :], axis=1)

    return pl.pallas_call(
        body,
        grid=(1,),
        in_specs=[pl.BlockSpec((N, F), lambda i: (0, 0)),
                  pl.BlockSpec((N,), lambda i: (0,)),
                  pl.BlockSpec((1, F), lambda i: (0, 0))],
        out_specs=pl.BlockSpec((B,), lambda i: (0,)),
        out_shape=jax.ShapeDtypeStruct((B,), jnp.float32),
    )(f, rep_atom, wread.reshape(1, F))


def kernel(z, xyz, nbrs, charge, spin, num_atoms, emb_z, q_vec, s_vec, W_g, W1, W2, W_out, w_read):
    src = nbrs[:, 0]
    dst = nbrs[:, 1]
    validf = (src != dst).astype(jnp.float32)

    grows = _GRP * _CHUNK
    npad = ((N + grows - 1) // grows) * grows
    zpad = jnp.concatenate([z, jnp.zeros((npad - N,), z.dtype)])
    emb_rows = _sc_gather(emb_z, zpad, F)[:N]
    x0 = _embed(emb_rows, charge, spin, q_vec, s_vec)

    tbl = jnp.concatenate(
        [xyz, _f32(z)[:, None], jnp.zeros((N, 12), jnp.float32)], axis=1)
    tsrc = _sc_gather(tbl, src, 16)
    tdst = _sc_gather(tbl, dst, 16)
    geoT, basT = _edge_geom(tsrc, tdst, validf)

    wg_pad = [jnp.concatenate([W_g[t], jnp.zeros((32 - K, F), jnp.float32)],
                              axis=0) for t in range(NCONV)]

    hs = []
    xes = []
    f = jnp.zeros((N, F), jnp.float32)
    x = x0
    for t in range(NCONV):
        xe = _sc_gather(x, src, F)
        xes.append(xe)
        msg = _msg(basT, xe, wg_pad[t])
        macc = _sc_scatter_add(msg, dst, N)
        x, h, f = _node(x, macc, f, W1[t], W2[t], W_out[t])
        hs.append(h)

    dBasT = jnp.zeros((32, E), jnp.float32)
    _, dU = _bwd_node(hs[2], W1[2], W2[2], W_out[2], w_read)
    dUe = _sc_gather(dU, dst, F)
    dBasT, pay = _bwd_edge(dUe, xes[2], basT, wg_pad[2], dBasT, True)
    scat = _sc_scatter_add(pay, src, N)

    g2, dU1 = _bwd_node(hs[1], W1[1], W2[1], W_out[1] + W_out[2], w_read,
                        dU_prev=dU, scat=scat, out_g=True)
    dUe = _sc_gather(dU1, dst, F)
    dBasT, pay = _bwd_edge(dUe, xes[1], basT, wg_pad[1], dBasT, True)
    scat = _sc_scatter_add(pay, src, N)

    _, dU0 = _bwd_node(hs[0], W1[0], W2[0], W_out[0], w_read,
                       dU_prev=dU1, scat=scat, g_prev=g2)
    dUe = _sc_gather(dU0, dst, F)
    dBasT = _bwd_edge(dUe, xes[0], basT, wg_pad[0], dBasT, False)[0]

    pay2 = _force_pay(tsrc, tdst, geoT, dBasT).reshape(2 * E, 16)
    facc = _sc_scatter_add(pay2, jnp.concatenate([dst, src]), N)
    fsum = facc[0] + facc[1]
    forces = fsum[:, :3]

    energy = _readout(f, fsum[:, 3], w_read)
    return energy, forces


# speedup vs baseline: 3.9662x; 1.1219x over previous
import functools

import jax
import jax.numpy as jnp
import numpy as np
from jax import lax
from jax.experimental import pallas as pl
from jax.experimental.pallas import tpu as pltpu
from jax.experimental.pallas import tpu_sc as plsc

N = 10000
B = 100
E = 160000
F = 128
MAXZ = 87
K = 20
NCONV = 3
RCUT = 5.0
GAMMA = 0.5

_CHUNK = 128
_NW = 32
_BE = 3200
_BN = 2000
_EH = E // 2

from math import lgamma as _lgamma

_LOGBIN = [_lgamma(K) - _lgamma(k + 1.0) - _lgamma(K - k) for k in range(K)]


def _f32(x):
    return x.astype(jnp.float32)


def _mesh():
    return plsc.VectorSubcoreMesh(core_axis_name="c", subcore_axis_name="s")


_GRP = 5


@functools.partial(jax.jit, static_argnames=("ncols",))
def _sc_gather(tbl, idx, ncols):
    M = idx.shape[0]
    nch = M // _CHUNK
    ngr = nch // _GRP
    gpw = (ngr + _NW - 1) // _NW
    grows = _GRP * _CHUNK
    idx2 = idx.reshape(nch, _CHUNK)

    def body(tbl_hbm, idx_hbm, out_hbm, idx_v, rows_v, semi, semg):
        wid = lax.axis_index("s") * 2 + lax.axis_index("c")

        def it(i, carry):
            gr = i * _NW + wid

            @pl.when(gr < ngr)
            def _():
                di = [pltpu.async_copy(idx_hbm.at[gr * _GRP + j],
                                       idx_v.at[j], semi)
                      for j in range(_GRP)]
                for dsc in di:
                    dsc.wait()
                descs = [pltpu.async_copy(
                    tbl_hbm.at[idx_v.at[j]],
                    rows_v.at[pl.ds(j * _CHUNK, _CHUNK)], semg)
                    for j in range(_GRP)]
                for dsc in descs:
                    dsc.wait()
                pltpu.sync_copy(rows_v, out_hbm.at[pl.ds(gr * grows, grows)])
            return carry

        lax.fori_loop(0, gpw, it, 0)

    return pl.kernel(
        body,
        out_type=jax.ShapeDtypeStruct((M, ncols), jnp.float32),
        mesh=_mesh(),
        scratch_types=[
            pltpu.VMEM((_GRP, _CHUNK), jnp.int32),
            pltpu.VMEM((grows, ncols), jnp.float32),
            pltpu.SemaphoreType.DMA,
            pltpu.SemaphoreType.DMA,
        ],
        compiler_params=pltpu.CompilerParams(use_tc_tiling_on_sc=(ncols % 128 == 0)),
    )(tbl, idx2)


@functools.partial(jax.jit, static_argnames=("nrows",))
def _sc_scatter_add(payload, idx, nrows):
    M, D = payload.shape
    nch = M // _CHUNK
    grp = _GRP if D <= 32 else (2 if nch % 2 == 0 else 1)
    ngr = nch // grp
    half = (ngr + 1) // 2
    niter = (half + 15) // 16
    grows = grp * _CHUNK
    nrows_pad = ((nrows + 127) // 128) * 128
    rows_sub = nrows_pad // 16
    zeros = jnp.zeros((nrows_pad, D), jnp.float32)
    idx2 = idx.reshape(nch, _CHUNK)

    def body(pay_hbm, idx_hbm, zeros_hbm, out_hbm, idx_v, rows_v, acc_sh,
             semi, semp, sema):
        cid = lax.axis_index("c")
        sid = lax.axis_index("s")
        r0 = sid * rows_sub
        pltpu.sync_copy(zeros_hbm.at[pl.ds(r0, rows_sub)],
                        acc_sh.at[pl.ds(r0, rows_sub)])
        plsc.subcore_barrier()

        def it(i, carry):
            gr = cid * half + i * 16 + sid

            @pl.when((i * 16 + sid < half) & (gr < ngr))
            def _():
                di = [pltpu.async_copy(idx_hbm.at[gr * grp + j],
                                       idx_v.at[j], semi)
                      for j in range(grp)]
                dp = pltpu.async_copy(pay_hbm.at[pl.ds(gr * grows, grows)],
                                      rows_v, semp)
                for dsc in di:
                    dsc.wait()
                dp.wait()
                descs = [pltpu.async_copy(
                    rows_v.at[pl.ds(j * _CHUNK, _CHUNK)],
                    acc_sh.at[idx_v.at[j]], sema, add=True)
                    for j in range(grp)]
                for dsc in descs:
                    dsc.wait()
            return carry

        lax.fori_loop(0, niter, it, 0)
        plsc.subcore_barrier()
        pltpu.sync_copy(acc_sh.at[pl.ds(r0, rows_sub)],
                        out_hbm.at[cid, pl.ds(r0, rows_sub)])

    out = pl.kernel(
        body,
        out_type=jax.ShapeDtypeStruct((2, nrows_pad, D), jnp.float32),
        mesh=_mesh(),
        scratch_types=[
            pltpu.VMEM((grp, _CHUNK), jnp.int32),
            pltpu.VMEM((grows, D), jnp.float32),
            pltpu.VMEM_SHARED((nrows_pad, D), jnp.float32),
            pltpu.SemaphoreType.DMA,
            pltpu.SemaphoreType.DMA,
            pltpu.SemaphoreType.DMA,
        ],
        compiler_params=pltpu.CompilerParams(use_tc_tiling_on_sc=(D % 128 == 0)),
    )(payload, idx2, zeros)
    return out[:, :nrows]


def _dot(a, b, dims):
    return lax.dot_general(a, b, (dims, ((), ())),
                           preferred_element_type=jnp.float32)


def _silu(h):
    return h * jax.nn.sigmoid(h)


def _silu_prime(h):
    s = jax.nn.sigmoid(h)
    return s * (1.0 + h * (1.0 - s))


def _edge_geom(tsrc, tdst, validf):
    M = tsrc.shape[0]
    neb = M // _BE

    def body(ts_ref, td_ref, va_ref, geo_ref, bas_ref):
        ts = ts_ref[...]
        td = td_ref[...]
        va = va_ref[...][0, 0]
        lane = lax.broadcasted_iota(jnp.int32, (1, 16), 1)
        m3 = (lane < 3).astype(jnp.float32)
        e3 = (lane == 3).astype(jnp.float32)
        dr = (td - ts) * m3
        d = jnp.sqrt(jnp.sum(dr * dr, axis=1) + 1e-12)
        zz = jnp.sum(ts * e3, axis=1) * jnp.sum(td * e3, axis=1)
        inside = (d < RCUT).astype(jnp.float32) * va
        fc = 0.5 * (jnp.cos(jnp.pi * d / RCUT) + 1.0) * inside
        fcp = -0.5 * (jnp.pi / RCUT) * jnp.sin(jnp.pi * d / RCUT) * inside
        u = jnp.exp(-GAMMA * d)
        c1u = jnp.clip(1.0 - u, 1e-10, 1.0)
        lu = jnp.log(jnp.clip(u, 1e-10, 1.0))
        l1u = jnp.log(c1u)
        urat = u / c1u
        zero = jnp.zeros_like(d)
        geo_ref[...] = jnp.concatenate(
            [v[None, :] for v in (d, fc, fcp, lu, l1u, urat, zz, zero)], axis=0)
        rows = [jnp.exp(_LOGBIN[k] + k * lu + (K - 1.0 - k) * l1u) * fc
                for k in range(K)] + [zero] * (32 - K)
        bas_ref[...] = jnp.concatenate([v[None, :] for v in rows], axis=0)

    return pl.pallas_call(
        body,
        grid=(neb,),
        in_specs=[pl.BlockSpec((_BE, 16), lambda i: (i, 0)),
                  pl.BlockSpec((_BE, 16), lambda i: (i, 0)),
                  pl.BlockSpec((1, 1, _BE), lambda i: (i, 0, 0))],
        out_specs=[pl.BlockSpec((8, _BE), lambda i: (0, i)),
                   pl.BlockSpec((32, _BE), lambda i: (0, i))],
        out_shape=[jax.ShapeDtypeStruct((8, M), jnp.float32),
                   jax.ShapeDtypeStruct((32, M), jnp.float32)],
    )(tsrc, tdst, validf.reshape(neb, 1, _BE))


def _msg(basT, xe, wg):
    M = xe.shape[0]

    def body(bas_ref, xe_ref, wg_ref, out_ref):
        g = _dot(bas_ref[...], wg_ref[...], ((0,), (0,)))
        out_ref[...] = xe_ref[...] * g

    return pl.pallas_call(
        body,
        grid=(M // _BE,),
        in_specs=[pl.BlockSpec((32, _BE), lambda i: (0, i)),
                  pl.BlockSpec((_BE, F), lambda i: (i, 0)),
                  pl.BlockSpec((32, F), lambda i: (0, 0))],
        out_specs=pl.BlockSpec((_BE, F), lambda i: (i, 0)),
        out_shape=jax.ShapeDtypeStruct((M, F), jnp.float32),
    )(basT, xe, wg)


def _node(x, m0, m1, f_in, w1, w2, wo):
    def body(x_ref, m0_ref, m1_ref, f_ref, w1_ref, w2_ref, wo_ref,
             xn_ref, h_ref, fo_ref):
        x = x_ref[...]
        m = m0_ref[0] + m0_ref[1] + m1_ref[0] + m1_ref[1]
        h = _dot(x + m, w1_ref[...], ((1,), (0,)))
        xn = x + _dot(_silu(h), w2_ref[...], ((1,), (0,)))
        xn_ref[...] = xn
        h_ref[...] = h
        fo_ref[...] = f_ref[...] + _dot(xn, wo_ref[...], ((1,), (0,)))

    nb = N // _BN
    pf = pl.BlockSpec((2, _BN, F), lambda i: (0, i, 0))
    return pl.pallas_call(
        body,
        grid=(nb,),
        in_specs=[pl.BlockSpec((_BN, F), lambda i: (i, 0)),
                  pf, pf,
                  pl.BlockSpec((_BN, F), lambda i: (i, 0)),
                  pl.BlockSpec((F, F), lambda i: (0, 0)),
                  pl.BlockSpec((F, F), lambda i: (0, 0)),
                  pl.BlockSpec((F, F), lambda i: (0, 0))],
        out_specs=[pl.BlockSpec((_BN, F), lambda i: (i, 0)),
                   pl.BlockSpec((_BN, F), lambda i: (i, 0)),
                   pl.BlockSpec((_BN, F), lambda i: (i, 0))],
        out_shape=[jax.ShapeDtypeStruct((N, F), jnp.float32)] * 3,
    )(x, m0, m1, f_in, w1, w2, wo)


def _bwd_node(h, w1, w2, wv, wread, dU_prev=None, scat0=None, scat1=None,
              g_prev=None, out_g=False):
    have_du = dU_prev is not None
    have_g = g_prev is not None

    def body(*refs):
        it = iter(refs)
        h_ref = next(it)
        w1_ref = next(it)
        w2_ref = next(it)
        wv_ref = next(it)
        wr_ref = next(it)
        du_ref = next(it) if have_du else None
        sc0_ref = next(it) if have_du else None
        sc1_ref = next(it) if have_du else None
        gp_ref = next(it) if have_g else None
        outs = list(it)
        v = _dot(wr_ref[...], wv_ref[...], ((1,), (1,)))
        g = jnp.broadcast_to(v, (_BN, F))
        if have_du:
            g = g + du_ref[...] + sc0_ref[0] + sc0_ref[1] \
                + sc1_ref[0] + sc1_ref[1]
        if have_g:
            g = g + gp_ref[...]
        dA = _dot(g, w2_ref[...], ((1,), (1,)))
        dU = _dot(dA * _silu_prime(h_ref[...]), w1_ref[...], ((1,), (1,)))
        if out_g:
            outs[0][...] = g
            outs[1][...] = dU
        else:
            outs[0][...] = dU

    nb = N // _BN
    nf = pl.BlockSpec((_BN, F), lambda i: (i, 0))
    ff = pl.BlockSpec((F, F), lambda i: (0, 0))
    pf = pl.BlockSpec((2, _BN, F), lambda i: (0, i, 0))
    in_specs = [nf, ff, ff, ff, pl.BlockSpec((1, F), lambda i: (0, 0))]
    args = [h, w1, w2, wv, wread.reshape(1, F)]
    if have_du:
        in_specs += [nf, pf, pf]
        args += [dU_prev, scat0, scat1]
    if have_g:
        in_specs += [nf]
        args += [g_prev]
    nout = 2 if out_g else 1
    out = pl.pallas_call(
        body,
        grid=(nb,),
        in_specs=in_specs,
        out_specs=[nf] * nout,
        out_shape=[jax.ShapeDtypeStruct((N, F), jnp.float32)] * nout,
    )(*args)
    return out if out_g else (None, out[0])


def _bwd_edge(dUe, xe, basT, wg, dBasT, with_pay):
    M = dUe.shape[0]

    def body(du_ref, xe_ref, bas_ref, wg_ref, dbin_ref, dbout_ref, *pay_ref):
        du = du_ref[...]
        q = du * xe_ref[...]
        dbout_ref[...] = dbin_ref[...] + _dot(wg_ref[...], q, ((1,), (1,)))
        if with_pay:
            g = _dot(bas_ref[...], wg_ref[...], ((0,), (0,)))
            pay_ref[0][...] = du * g

    ef = pl.BlockSpec((_BE, F), lambda i: (i, 0))
    bs = pl.BlockSpec((32, _BE), lambda i: (0, i))
    outs = [jax.ShapeDtypeStruct((32, M), jnp.float32)]
    out_specs = [bs]
    if with_pay:
        outs.append(jax.ShapeDtypeStruct((M, F), jnp.float32))
        out_specs.append(ef)
    return pl.pallas_call(
        body,
        grid=(M // _BE,),
        in_specs=[ef, ef, bs, pl.BlockSpec((32, F), lambda i: (0, 0)), bs],
        out_specs=out_specs,
        out_shape=outs,
        input_output_aliases={4: 0},
    )(dUe, xe, basT, wg, dBasT)


def _force_pay(tsrc, tdst, geoT, dBasT):
    M = tsrc.shape[0]

    def body(ts_ref, td_ref, geo_ref, db_ref, out_ref):
        geo = geo_ref[...]
        db = db_ref[...]
        d, fc, fcp = geo[0], geo[1], geo[2]
        lu, l1u, urat, zz = geo[3], geo[4], geo[5], geo[6]
        acc = jnp.zeros_like(d)
        for k in range(K):
            bern = jnp.exp(_LOGBIN[k] + k * lu + (K - 1.0 - k) * l1u)
            bp = bern * (GAMMA * (-float(k) + (K - 1.0 - k) * urat))
            acc = acc + db[k] * (bp * fc + bern * fcp)
        dm = jnp.maximum(d, 1e-3)
        drep = 0.5 * zz * (-(d > 1e-3).astype(jnp.float32) / (dm * dm) * fc
                           + fcp / dm)
        coef = (acc + drep) / d
        rep = zz / dm * fc
        lane = lax.broadcasted_iota(jnp.int32, (1, 16), 1)
        m3 = (lane < 3).astype(jnp.float32)
        e3 = (lane == 3).astype(jnp.float32)
        dr = (td_ref[...] - ts_ref[...]) * m3
        fv = dr * coef[:, None]
        out_ref[0] = fv + rep[:, None] * e3
        out_ref[1] = -fv

    return pl.pallas_call(
        body,
        grid=(M // _BE,),
        in_specs=[pl.BlockSpec((_BE, 16), lambda i: (i, 0)),
                  pl.BlockSpec((_BE, 16), lambda i: (i, 0)),
                  pl.BlockSpec((8, _BE), lambda i: (0, i)),
                  pl.BlockSpec((32, _BE), lambda i: (0, i))],
        out_specs=pl.BlockSpec((2, _BE, 16), lambda i: (0, i, 0)),
        out_shape=jax.ShapeDtypeStruct((2, M, 16), jnp.float32),
    )(tsrc, tdst, geoT, dBasT)


def _embed(emb_rows, charge, spin, q_vec, s_vec):
    bn = 1000
    nb = N // bn

    def body(e_ref, c_ref, s_ref, q_ref, sv_ref, out_ref):
        cs = c_ref[...][:, None] * q_ref[...][None, :] \
            + s_ref[...][:, None] * sv_ref[...][None, :]
        i = pl.program_id(0)
        jj = (lax.broadcasted_iota(jnp.int32, (bn, B), 0) + i * bn) // (N // B)
        ii = lax.broadcasted_iota(jnp.int32, (bn, B), 1)
        sel = (jj == ii).astype(jnp.float32)
        out_ref[...] = e_ref[...] + _dot(sel, cs, ((1,), (0,)))

    return pl.pallas_call(
        body,
        grid=(nb,),
        in_specs=[pl.BlockSpec((bn, F), lambda i: (i, 0)),
                  pl.BlockSpec((B,), lambda i: (0,)),
                  pl.BlockSpec((B,), lambda i: (0,)),
                  pl.BlockSpec((F,), lambda i: (0,)),
                  pl.BlockSpec((F,), lambda i: (0,))],
        out_specs=pl.BlockSpec((bn, F), lambda i: (i, 0)),
        out_shape=jax.ShapeDtypeStruct((N, F), jnp.float32),
    )(emb_rows, charge, spin, q_vec, s_vec)


def _readout(f, rep_atom, wread):
    def body(f_ref, r_ref, w_ref, out_ref):
        e = _dot(f_ref[...], w_ref[...], ((1,), (1,)))[:, 0] + 0.5 * r_ref[...]
        ii = lax.broadcasted_iota(jnp.int32, (B, N), 0)
        jj = lax.broadcasted_iota(jnp.int32, (B, N), 1) // (N // B)
        sel = (ii == jj).astype(jnp.float32)
        out_ref[...] = jnp.sum(sel * e[None, :], axis=1)

    return pl.pallas_call(
        body,
        grid=(1,),
        in_specs=[pl.BlockSpec((N, F), lambda i: (0, 0)),
                  pl.BlockSpec((N,), lambda i: (0,)),
                  pl.BlockSpec((1, F), lambda i: (0, 0))],
        out_specs=pl.BlockSpec((B,), lambda i: (0,)),
        out_shape=jax.ShapeDtypeStruct((B,), jnp.float32),
    )(f, rep_atom, wread.reshape(1, F))


def kernel(z, xyz, nbrs, charge, spin, num_atoms, emb_z, q_vec, s_vec, W_g, W1, W2, W_out, w_read):
    src = nbrs[:, 0]
    dst = nbrs[:, 1]
    validf = (src != dst).astype(jnp.float32)
    srcs = [src[:_EH], src[_EH:]]
    dsts = [dst[:_EH], dst[_EH:]]
    vals = [validf[:_EH], validf[_EH:]]

    grows = _GRP * _CHUNK
    npad = ((N + grows - 1) // grows) * grows
    zpad = jnp.concatenate([z, jnp.zeros((npad - N,), z.dtype)])
    emb_rows = _sc_gather(emb_z, zpad, F)[:N]
    x0 = _embed(emb_rows, charge, spin, q_vec, s_vec)

    tbl = jnp.concatenate(
        [xyz, _f32(z)[:, None], jnp.zeros((N, 12), jnp.float32)], axis=1)
    tsrc = [_sc_gather(tbl, s, 16) for s in srcs]
    tdst = [_sc_gather(tbl, d, 16) for d in dsts]
    gb = [_edge_geom(tsrc[i], tdst[i], vals[i]) for i in range(2)]
    geoT = [g[0] for g in gb]
    basT = [g[1] for g in gb]

    wg_pad = [jnp.concatenate([W_g[t], jnp.zeros((32 - K, F), jnp.float32)],
                              axis=0) for t in range(NCONV)]

    hs = []
    xes = []
    f = jnp.zeros((N, F), jnp.float32)
    x = x0
    for t in range(NCONV):
        xe = [_sc_gather(x, srcs[i], F) for i in range(2)]
        xes.append(xe)
        sc = [_sc_scatter_add(_msg(basT[i], xe[i], wg_pad[t]), dsts[i], N)
              for i in range(2)]
        x, h, f = _node(x, sc[0], sc[1], f, W1[t], W2[t], W_out[t])
        hs.append(h)

    dBasT = [jnp.zeros((32, _EH), jnp.float32) for _ in range(2)]

    def bwd_layer(dU, t, with_pay):
        scat = [None, None]
        for i in range(2):
            dUe = _sc_gather(dU, dsts[i], F)
            out = _bwd_edge(dUe, xes[t][i], basT[i], wg_pad[t],
                            dBasT[i], with_pay)
            dBasT[i] = out[0]
            if with_pay:
                scat[i] = _sc_scatter_add(out[1], srcs[i], N)
        return scat

    _, dU = _bwd_node(hs[2], W1[2], W2[2], W_out[2], w_read)
    scat = bwd_layer(dU, 2, True)

    g2, dU1 = _bwd_node(hs[1], W1[1], W2[1], W_out[1] + W_out[2], w_read,
                        dU_prev=dU, scat0=scat[0], scat1=scat[1], out_g=True)
    scat = bwd_layer(dU1, 1, True)

    _, dU0 = _bwd_node(hs[0], W1[0], W2[0], W_out[0], w_read,
                       dU_prev=dU1, scat0=scat[0], scat1=scat[1], g_prev=g2)
    bwd_layer(dU0, 0, False)

    facc = [_sc_scatter_add(
        _force_pay(tsrc[i], tdst[i], geoT[i], dBasT[i]).reshape(2 * _EH, 16),
        jnp.concatenate([dsts[i], srcs[i]]), N) for i in range(2)]
    fsum = facc[0][0] + facc[0][1] + facc[1][0] + facc[1][1]
    forces = fsum[:, :3]

    energy = _readout(f, fsum[:, 3], w_read)
    return energy, forces
